# Initial kernel scaffold; baseline (speedup 1.0000x reference)
import jax
import jax.numpy as jnp
from jax.experimental import pallas as pl


def kernel(start, memory, length):
    raise NotImplementedError("write your pallas kernel here")


# speedup vs baseline: 1.1107x; 1.1107x over previous
import functools

import jax
import jax.numpy as jnp
from jax import lax
from jax.experimental import pallas as pl
from jax.experimental.pallas import tpu as pltpu
from jax.experimental.pallas import tpu_sc as plsc

BITS = 20
STEPS = 512
ROW = 32
TBL = 1 << BITS


def _walk_body(start_hbm, c_hbm, mem_hbm, out_hbm, st_v, idx_v, out_v, c_v, sem):
    cid = lax.axis_index("c")
    sid = lax.axis_index("s")

    @pl.when(jnp.logical_and(cid == 0, sid == 0))
    def _():
        pltpu.sync_copy(start_hbm, st_v)
        pltpu.sync_copy(c_hbm, c_v)
        cv = c_v[...]

        lane = lax.iota(jnp.int32, 16)
        hi = lane < (BITS - 16)
        zero = lane * 0
        pow0 = 1 << lane
        pow1 = jnp.where(hi, 1 << (lane + 16), zero)
        off0 = lane * TBL
        off1 = jnp.where(hi, (lane + 16) * TBL, zero)

        dnums = lax.GatherDimensionNumbers(
            offset_dims=(), collapsed_slice_dims=(0,), start_index_map=(0,))

        def splat_sum(x):
            for k in (8, 4, 2, 1):
                perm = lane ^ k
                x = x + lax.gather(x, perm[:, None], dimension_numbers=dnums,
                                   slice_sizes=(1,),
                                   mode=lax.GatherScatterMode.PROMISE_IN_BOUNDS)
            return x

        def to_addr(w0, w1):
            return splat_sum(jnp.where(w0 > 0.5, pow0, zero)
                             + jnp.where(w1 > 0.5, pow1, zero))

        v0 = st_v[pl.ds(0, 16)]
        v1 = st_v[pl.ds(16, 16)]
        out_v[0, pl.ds(0, 16)] = v0 + cv
        out_v[0, pl.ds(16, 16)] = v1 + cv
        a0 = to_addr(v0, v1)

        def step(t, addr):
            idx_v[pl.ds(0, 16)] = addr + off0
            idx_v[pl.ds(16, 16)] = addr + off1
            pltpu.async_copy(mem_hbm.at[idx_v], out_v.at[t], sem).wait()
            w0 = out_v[t, pl.ds(0, 16)]
            w1 = out_v[t, pl.ds(16, 16)]
            out_v[t, pl.ds(0, 16)] = w0 + cv
            out_v[t, pl.ds(16, 16)] = w1 + cv
            return to_addr(w0, w1)

        lax.fori_loop(1, STEPS, step, a0)
        pltpu.sync_copy(out_v, out_hbm)


@functools.partial(jax.jit, static_argnames=())
def _walk(start32, cvec, mem_flat):
    mesh = plsc.VectorSubcoreMesh(core_axis_name="c", subcore_axis_name="s")
    return pl.kernel(
        _walk_body,
        out_type=jax.ShapeDtypeStruct((STEPS, ROW), jnp.float32),
        mesh=mesh,
        scratch_types=[
            pltpu.VMEM((32,), jnp.float32),
            pltpu.VMEM((32,), jnp.int32),
            pltpu.VMEM((STEPS, ROW), jnp.float32),
            pltpu.VMEM((16,), jnp.float32),
            pltpu.SemaphoreType.DMA,
        ],
    )(start32, cvec, mem_flat)


def kernel(start, memory, length):
    start32 = jnp.zeros((32,), jnp.float32).at[:BITS].set(start)
    c = (jnp.asarray(length, jnp.int32) - STEPS).astype(jnp.float32)
    cvec = jnp.full((16,), c, jnp.float32)
    out = _walk(start32, cvec, memory.reshape(-1))
    return out[:, :BITS]


# speedup vs baseline: 1.2111x; 1.0904x over previous
import functools

import jax
import jax.numpy as jnp
from jax import lax
from jax.experimental import pallas as pl
from jax.experimental.pallas import tpu as pltpu
from jax.experimental.pallas import tpu_sc as plsc

BITS = 20
STEPS = 512
TBL = 1 << BITS
BLK = 8192
ROW = 32
CROW = 16


def _tbl_body(mem_ref, t_ref):
    m = mem_ref[...]
    pw = 1 << lax.broadcasted_iota(jnp.int32, (BITS, BLK), 0)
    t_ref[...] = jnp.sum(jnp.where(m > 0.5, pw, 0), axis=0, keepdims=True)


_build_table = pl.pallas_call(
    _tbl_body,
    grid=(TBL // BLK,),
    in_specs=[pl.BlockSpec((BITS, BLK), lambda i: (0, i))],
    out_specs=pl.BlockSpec((1, BLK), lambda i: (0, i)),
    out_shape=jax.ShapeDtypeStruct((1, TBL), jnp.int32),
)


def _walk_body(start_hbm, c_hbm, t_hbm, mem_hbm, out_hbm,
               t_sp, ab_sp, st_v, c_v, ab_v, av_v, idx_v, ob_v, sem):
    cid = lax.axis_index("c")
    sid = lax.axis_index("s")
    gw = cid * 16 + sid

    lane = lax.iota(jnp.int32, 16)
    hi = lane < (BITS - 16)
    zero = lane * 0
    pow0 = 1 << lane
    pow1 = jnp.where(hi, 1 << (lane + 16), zero)
    off0 = lane * TBL
    off1 = jnp.where(hi, (lane + 16) * TBL, zero)

    pltpu.sync_copy(c_hbm, c_v)

    seg = TBL // 16
    pltpu.sync_copy(t_hbm.at[pl.ds(sid * seg, seg)],
                    t_sp.at[pl.ds(sid * seg, seg)])
    plsc.subcore_barrier()

    @pl.when(sid == 0)
    def _():
        pltpu.sync_copy(start_hbm, st_v)

        dnums = lax.GatherDimensionNumbers(
            offset_dims=(), collapsed_slice_dims=(0,), start_index_map=(0,))

        def splat_sum(x):
            for k in (8, 4, 2, 1):
                perm = lane ^ k
                x = x + lax.gather(x, perm[:, None], dimension_numbers=dnums,
                                   slice_sizes=(1,),
                                   mode=lax.GatherScatterMode.PROMISE_IN_BOUNDS)
            return x

        v0 = st_v[pl.ds(0, 16)]
        v1 = st_v[pl.ds(16, 16)]
        a0 = splat_sum(jnp.where(v0 > 0.5, pow0, zero)
                       + jnp.where(v1 > 0.5, pow1, zero))
        ab_v[pl.ds(0, 16)] = a0
        ab_v[pl.ds(16, 16)] = a0

        def chain(t, carry):
            pltpu.async_copy(
                t_sp.at[ab_v.at[pl.ds(CROW * (t - 1), CROW)]],
                ab_v.at[pl.ds(CROW * t, CROW)], sem).wait()
            return carry

        lax.fori_loop(2, STEPS, chain, 0)
        pltpu.sync_copy(ab_v, ab_sp)

    plsc.subcore_barrier()

    pltpu.sync_copy(ab_sp.at[pl.ds(256 * gw, 256)], av_v)
    for r in range(16):
        a = av_v[pl.ds(CROW * r, 16)]
        idx_v[pl.ds(ROW * r, 16)] = a + off0
        idx_v[pl.ds(ROW * r + 16, 16)] = a + off1
    cps = [pltpu.async_copy(mem_hbm.at[idx_v.at[pl.ds(128 * k, 128)]],
                            ob_v.at[pl.ds(128 * k, 128)], sem)
           for k in range(4)]
    for cp in cps:
        cp.wait()

    cv = c_v[...]
    for j in range(ROW):
        ob_v[pl.ds(16 * j, 16)] = ob_v[pl.ds(16 * j, 16)] + cv

    @pl.when(gw == 0)
    def _():
        ob_v[pl.ds(0, 16)] = st_v[pl.ds(0, 16)] + cv
        ob_v[pl.ds(16, 16)] = st_v[pl.ds(16, 16)] + cv

    pltpu.sync_copy(ob_v, out_hbm.at[pl.ds(512 * gw, 512)])


@jax.jit
def _sc_walk(start32, cvec, t_flat, mem_flat):
    mesh = plsc.VectorSubcoreMesh(core_axis_name="c", subcore_axis_name="s")
    return pl.kernel(
        _walk_body,
        out_type=jax.ShapeDtypeStruct((STEPS * ROW,), jnp.float32),
        mesh=mesh,
        scratch_types=[
            pltpu.VMEM_SHARED((TBL,), jnp.int32),
            pltpu.VMEM_SHARED((STEPS * CROW,), jnp.int32),
            pltpu.VMEM((32,), jnp.float32),
            pltpu.VMEM((16,), jnp.float32),
            pltpu.VMEM((STEPS * CROW,), jnp.int32),
            pltpu.VMEM((16 * CROW,), jnp.int32),
            pltpu.VMEM((16 * ROW,), jnp.int32),
            pltpu.VMEM((16 * ROW,), jnp.float32),
            pltpu.SemaphoreType.DMA,
        ],
    )(start32, cvec, t_flat, mem_flat)


def kernel(start, memory, length):
    start32 = jnp.zeros((32,), jnp.float32).at[:BITS].set(start)
    c = (jnp.asarray(length, jnp.int32) - STEPS).astype(jnp.float32)
    cvec = jnp.full((16,), c, jnp.float32)
    t_flat = _build_table(memory).reshape(-1)
    out = _sc_walk(start32, cvec, t_flat, memory.reshape(-1))
    return out.reshape(STEPS, ROW)[:, :BITS]


# speedup vs baseline: 1.2144x; 1.0027x over previous
import functools

import jax
import jax.numpy as jnp
from jax import lax
from jax.experimental import pallas as pl
from jax.experimental.pallas import tpu as pltpu
from jax.experimental.pallas import tpu_sc as plsc

BITS = 20
STEPS = 512
TBL = 1 << BITS
BLK = 8192
ROW = 32
CROW = 16


def _tbl_body(mem_ref, t_ref):
    m = mem_ref[...]
    pw = 1 << lax.broadcasted_iota(jnp.int32, (BITS, BLK), 0)
    t_ref[...] = jnp.sum(jnp.where(m > 0.5, pw, 0), axis=0, keepdims=True)


_build_table = pl.pallas_call(
    _tbl_body,
    grid=(TBL // BLK,),
    in_specs=[pl.BlockSpec((BITS, BLK), lambda i: (0, i))],
    out_specs=pl.BlockSpec((1, BLK), lambda i: (0, i)),
    out_shape=jax.ShapeDtypeStruct((1, TBL), jnp.int32),
)


def _walk_body(start_hbm, c_hbm, t_hbm, mem_hbm, out_hbm,
               t_sp, ab_sp, st_v, c_v, ab_v, av_v, idx_v, ob_v, tmp_v, sem):
    cid = lax.axis_index("c")
    sid = lax.axis_index("s")
    gw = cid * 16 + sid

    lane = lax.iota(jnp.int32, 16)
    hi = lane < (BITS - 16)
    zero = lane * 0
    pow0 = 1 << lane
    pow1 = jnp.where(hi, 1 << (lane + 16), zero)
    off0 = lane * TBL
    off1 = jnp.where(hi, (lane + 16) * TBL, zero)

    pltpu.sync_copy(c_hbm, c_v)

    seg = TBL // 16
    pltpu.sync_copy(t_hbm.at[pl.ds(sid * seg, seg)],
                    t_sp.at[pl.ds(sid * seg, seg)])
    plsc.subcore_barrier()

    @pl.when(sid == 0)
    def _():
        pltpu.sync_copy(start_hbm, st_v)

        dnums = lax.GatherDimensionNumbers(
            offset_dims=(), collapsed_slice_dims=(0,), start_index_map=(0,))

        def splat_sum(x):
            for k in (8, 4, 2, 1):
                perm = lane ^ k
                x = x + lax.gather(x, perm[:, None], dimension_numbers=dnums,
                                   slice_sizes=(1,),
                                   mode=lax.GatherScatterMode.PROMISE_IN_BOUNDS)
            return x

        v0 = st_v[pl.ds(0, 16)]
        v1 = st_v[pl.ds(16, 16)]
        a0 = splat_sum(jnp.where(v0 > 0.5, pow0, zero)
                       + jnp.where(v1 > 0.5, pow1, zero))
        ab_v[pl.ds(0, 16)] = a0
        ab_v[pl.ds(16, 16)] = a0

        def chain(t, a):
            base = pl.multiple_of(a & ~15, 8)
            pltpu.sync_copy(t_sp.at[pl.ds(base, 16)], tmp_v)
            v = tmp_v[...]
            lsel = jnp.broadcast_to(a & 15, (16,))
            nxt = lax.gather(v, lsel[:, None], dimension_numbers=dnums,
                             slice_sizes=(1,),
                             mode=lax.GatherScatterMode.PROMISE_IN_BOUNDS)
            ab_v[pl.ds(CROW * t, 16)] = nxt
            return jnp.where(lane == 0, nxt, zero)[0]

        lax.fori_loop(2, STEPS, chain, jnp.where(lane == 0, a0, zero)[0])
        pltpu.sync_copy(ab_v, ab_sp)

    plsc.subcore_barrier()

    pltpu.sync_copy(ab_sp.at[pl.ds(256 * gw, 256)], av_v)
    for r in range(16):
        a = av_v[pl.ds(CROW * r, 16)]
        idx_v[pl.ds(ROW * r, 16)] = a + off0
        idx_v[pl.ds(ROW * r + 16, 16)] = a + off1
    cps = [pltpu.async_copy(mem_hbm.at[idx_v.at[pl.ds(128 * k, 128)]],
                            ob_v.at[pl.ds(128 * k, 128)], sem)
           for k in range(4)]
    for cp in cps:
        cp.wait()

    cv = c_v[...]
    for j in range(ROW):
        ob_v[pl.ds(16 * j, 16)] = ob_v[pl.ds(16 * j, 16)] + cv

    @pl.when(gw == 0)
    def _():
        ob_v[pl.ds(0, 16)] = st_v[pl.ds(0, 16)] + cv
        ob_v[pl.ds(16, 16)] = st_v[pl.ds(16, 16)] + cv

    pltpu.sync_copy(ob_v, out_hbm.at[pl.ds(512 * gw, 512)])


@jax.jit
def _sc_walk(start32, cvec, t_flat, mem_flat):
    mesh = plsc.VectorSubcoreMesh(core_axis_name="c", subcore_axis_name="s")
    return pl.kernel(
        _walk_body,
        out_type=jax.ShapeDtypeStruct((STEPS * ROW,), jnp.float32),
        mesh=mesh,
        scratch_types=[
            pltpu.VMEM_SHARED((TBL,), jnp.int32),
            pltpu.VMEM_SHARED((STEPS * CROW,), jnp.int32),
            pltpu.VMEM((32,), jnp.float32),
            pltpu.VMEM((16,), jnp.float32),
            pltpu.VMEM((STEPS * CROW,), jnp.int32),
            pltpu.VMEM((16 * CROW,), jnp.int32),
            pltpu.VMEM((16 * ROW,), jnp.int32),
            pltpu.VMEM((16 * ROW,), jnp.float32),
            pltpu.VMEM((16,), jnp.int32),
            pltpu.SemaphoreType.DMA,
        ],
    )(start32, cvec, t_flat, mem_flat)


def kernel(start, memory, length):
    start32 = jnp.zeros((32,), jnp.float32).at[:BITS].set(start)
    c = (jnp.asarray(length, jnp.int32) - STEPS).astype(jnp.float32)
    cvec = jnp.full((16,), c, jnp.float32)
    t_flat = _build_table(memory).reshape(-1)
    out = _sc_walk(start32, cvec, t_flat, memory.reshape(-1))
    return out.reshape(STEPS, ROW)[:, :BITS]
